# baseline (speedup 1.0000x reference)
import functools

import jax
import jax.numpy as jnp
from jax import lax
from jax.experimental import pallas as pl
from jax.experimental.pallas import tpu as pltpu
from jax.experimental.pallas import tpu_sc as plsc

_N_TOKENS = 10
_B = 1024
_L = 210
_D = 64
_CONTENT = _L - _N_TOKENS
_NUM_WORKERS = 32
_VOCAB2 = 2_000_000
_HALF = _B // 2
_LANES = 16


def _soft_embedding_sc(tok2, wte2, learned_embedding):
    mesh = plsc.VectorSubcoreMesh(core_axis_name="c", subcore_axis_name="s")

    @functools.partial(
        pl.kernel,
        mesh=mesh,
        out_type=jax.ShapeDtypeStruct((_L, _D, _B), jnp.float32),
        scratch_types=[
            pltpu.VMEM((_B,), jnp.int32),
            pltpu.VMEM((_HALF, _D), jnp.float32),
            pltpu.VMEM((_D, _B), jnp.float32),
            pltpu.VMEM((_N_TOKENS, _D), jnp.float32),
            pltpu.SemaphoreType.DMA,
        ],
        compiler_params=pltpu.CompilerParams(
            use_tc_tiling_on_sc=False, needs_layout_passes=False
        ),
    )
    def k(tok_hbm, wte_hbm, le_hbm, out_hbm, idx_v, gbuf, tbuf, le_v, sem):
        wid = lax.axis_index("s") * 2 + lax.axis_index("c")

        @pl.when(wid < _N_TOKENS)
        def _():
            pltpu.sync_copy(le_hbm, le_v)
            row_sel = jnp.full((_LANES,), wid, jnp.int32)

            def fill_row(d, carry):
                splat = plsc.load_gather(
                    le_v, [row_sel, jnp.full((_LANES,), d, jnp.int32)]
                )

                def store8(h, c2):
                    for j in range(8):
                        tbuf[d, pl.ds((h * 8 + j) * _LANES, _LANES)] = splat
                    return c2

                lax.fori_loop(0, _B // (_LANES * 8), store8, 0)
                return carry

            lax.fori_loop(0, _D, fill_row, 0)
            pltpu.sync_copy(tbuf, out_hbm.at[wid])

        iota = lax.iota(jnp.int32, _LANES)

        def do_plane(li):
            pltpu.sync_copy(tok_hbm.at[_N_TOKENS + li], idx_v)
            for half in range(2):
                pltpu.async_copy(
                    wte_hbm.at[idx_v.at[pl.ds(half * _HALF, _HALF)]],
                    gbuf,
                    sem,
                ).wait()

                def tr_d(d, carry):
                    col = jnp.full((_LANES,), d, jnp.int32)

                    def tr_h(h, c2):
                        for j in range(4):
                            rows = iota + (h * 4 + j) * _LANES
                            vals = plsc.load_gather(gbuf, [rows, col])
                            tbuf[
                                d,
                                pl.ds(
                                    half * _HALF + (h * 4 + j) * _LANES,
                                    _LANES,
                                ),
                            ] = vals
                        return c2

                    lax.fori_loop(0, _HALF // (_LANES * 4), tr_h, 0)
                    return carry

                lax.fori_loop(0, _D, tr_d, 0)
            pltpu.sync_copy(tbuf, out_hbm.at[_N_TOKENS + li])

        for i in range((_CONTENT + _NUM_WORKERS - 1) // _NUM_WORKERS):
            li = wid + i * _NUM_WORKERS

            @pl.when(li < _CONTENT)
            def _():
                do_plane(li)

    return k(tok2, wte2, learned_embedding)


def kernel(tokens, wte, learned_embedding):
    wte2 = jnp.pad(wte, ((0, 0), (0, _D))).reshape(_VOCAB2, _D)
    tok2 = (tokens * 2).T
    out = _soft_embedding_sc(tok2, wte2, learned_embedding)
    return jnp.transpose(out, (2, 0, 1))


# speedup vs baseline: 1.3921x; 1.3921x over previous
import functools

import jax
import jax.numpy as jnp
from jax import lax
from jax.experimental import pallas as pl
from jax.experimental.pallas import tpu as pltpu
from jax.experimental.pallas import tpu_sc as plsc

_N_TOKENS = 10
_B = 1024
_L = 210
_D = 64
_CONTENT = _L - _N_TOKENS
_NUM_WORKERS = 32
_BPW = _B // _NUM_WORKERS
_G = 4
_NCHUNK = _BPW // _G
_NBUF = 2
_VOCAB2 = 2_000_000


def _soft_embedding_sc(idx, wte2, learned_embedding):
    mesh = plsc.VectorSubcoreMesh(core_axis_name="c", subcore_axis_name="s")

    @functools.partial(
        pl.kernel,
        mesh=mesh,
        out_type=jax.ShapeDtypeStruct((_B, _L, _D), jnp.float32),
        scratch_types=[
            pltpu.VMEM((_BPW, _CONTENT), jnp.int32),
            pltpu.VMEM((_NBUF, _G, _L, _D), jnp.float32),
            pltpu.SemaphoreType.DMA,
            pltpu.SemaphoreType.DMA,
            pltpu.SemaphoreType.DMA,
            pltpu.SemaphoreType.DMA,
        ],
        compiler_params=pltpu.CompilerParams(use_tc_tiling_on_sc=False),
    )
    def k(idx_hbm, wte_hbm, le_hbm, out_hbm, idx_v, bufs, sg0, sg1, sw0, sw1):
        wid = lax.axis_index("s") * 2 + lax.axis_index("c")
        base = wid * _BPW
        sem_g = (sg0, sg1)
        sem_w = (sw0, sw1)

        pltpu.sync_copy(idx_hbm.at[pl.ds(base, _BPW)], idx_v)
        for p in range(_NBUF):
            for b in range(_G):
                pltpu.sync_copy(le_hbm, bufs.at[p, b, pl.ds(0, _N_TOKENS)])

        def gather_descs(g, p, issue):
            for b in range(_G):
                src = wte_hbm.at[idx_v.at[g * _G + b]]
                dst = bufs.at[p, b, pl.ds(_N_TOKENS, _CONTENT)]
                if issue:
                    pltpu.async_copy(src, dst, sem_g[p])
                else:
                    pltpu.make_async_copy(src, dst, sem_g[p]).wait()

        def write_desc(g, p, issue):
            src = bufs.at[p]
            dst = out_hbm.at[pl.ds(base + g * _G, _G)]
            if issue:
                pltpu.async_copy(src, dst, sem_w[p])
            else:
                pltpu.make_async_copy(src, dst, sem_w[p]).wait()

        gather_descs(0, 0, True)
        gather_descs(1, 1, True)

        def body(g2, carry):
            for p in range(_NBUF):
                g = g2 * _NBUF + p
                gather_descs(g, p, False)
                write_desc(g, p, True)

            @pl.when(g2 < _NCHUNK // _NBUF - 1)
            def _():
                for p in range(_NBUF):
                    g = g2 * _NBUF + p
                    write_desc(g, p, False)
                    gather_descs(g + _NBUF, p, True)

            return carry

        lax.fori_loop(0, _NCHUNK // _NBUF, body, 0)
        write_desc(_NCHUNK - 2, 0, False)
        write_desc(_NCHUNK - 1, 1, False)

    return k(idx, wte2, learned_embedding)


def kernel(tokens, wte, learned_embedding):
    wte2 = jnp.pad(wte, ((0, 0), (0, _D))).reshape(_VOCAB2, _D)
    idx = tokens[:, _N_TOKENS:] * 2
    return _soft_embedding_sc(idx, wte2, learned_embedding)
